# Initial kernel scaffold; baseline (speedup 1.0000x reference)
import jax
import jax.numpy as jnp
from jax.experimental import pallas as pl


def kernel(x, edge_index, W, b, gamma, beta):
    raise NotImplementedError("write your pallas kernel here")


# speedup vs baseline: 34.9194x; 34.9194x over previous
import functools

import jax
import jax.numpy as jnp
from jax import lax
from jax.experimental import pallas as pl
from jax.experimental.pallas import tpu as pltpu
from jax.experimental.pallas import tpu_sc as plsc

N = 10000
E = 320000
D = 128
NC = 2
NS = 16
NW = NC * NS
EPW = E // NW
CHUNK = 50
NITER = EPW // CHUNK
IBLK = 8
NPAD = 10240
RPT = NPAD // NS

_mesh = plsc.VectorSubcoreMesh(
    core_axis_name="c", subcore_axis_name="s", num_cores=NC, num_subcores=NS)


@functools.partial(
    pl.kernel,
    out_type=jax.ShapeDtypeStruct((NC, NPAD), jnp.float32),
    mesh=_mesh,
    scratch_types=[
        pltpu.VMEM((NITER, CHUNK), jnp.int32),
        pltpu.VMEM((64,), jnp.float32),
        pltpu.VMEM((RPT,), jnp.float32),
        pltpu.VMEM_SHARED((NPAD,), jnp.float32),
        pltpu.SemaphoreType.DMA,
    ],
)
def _deg_kernel(col_hbm, deg_hbm, col_v, ones_v, zf_v, deg_sh, sem):
    c = lax.axis_index("c")
    s = lax.axis_index("s")
    wid = c * NS + s
    pltpu.sync_copy(col_hbm.at[wid], col_v)
    for j in range(64 // 16):
        ones_v[pl.ds(j * 16, 16)] = jnp.ones((16,), jnp.float32)
    for j in range(RPT // 16):
        zf_v[pl.ds(j * 16, 16)] = jnp.zeros((16,), jnp.float32)
    pltpu.sync_copy(zf_v, deg_sh.at[pl.ds(s * RPT, RPT)])
    plsc.subcore_barrier()
    ones_c = ones_v.at[pl.ds(0, CHUNK)]
    descs = []
    for i in range(NITER):
        descs.append(
            pltpu.async_copy(ones_c, deg_sh.at[col_v.at[i]], sem, add=True))
    for d in descs:
        d.wait()
    plsc.subcore_barrier()
    pltpu.sync_copy(deg_sh.at[pl.ds(s * RPT, RPT)],
                    deg_hbm.at[c, pl.ds(s * RPT, RPT)])


@functools.partial(
    pl.kernel,
    out_type=jax.ShapeDtypeStruct((NC, N, D), jnp.float32),
    mesh=_mesh,
    scratch_types=[
        pltpu.VMEM((2, IBLK, CHUNK), jnp.int32),
        pltpu.VMEM((2, IBLK, CHUNK), jnp.int32),
        pltpu.VMEM((CHUNK, D), jnp.float32),
        pltpu.VMEM((CHUNK, D), jnp.float32),
        pltpu.VMEM((16, D), jnp.float32),
        pltpu.VMEM_SHARED((NPAD, D), jnp.float32),
        pltpu.SemaphoreType.DMA,
        pltpu.SemaphoreType.DMA,
        pltpu.SemaphoreType.DMA,
        pltpu.SemaphoreType.DMA,
    ],
)
def _msg_kernel(row_hbm, col_hbm, hp_hbm, out_hbm,
                row_v, col_v, buf0, buf1, zb, acc_sh, sem0, sem1, semr, semc):
    c = lax.axis_index("c")
    s = lax.axis_index("s")
    wid = c * NS + s
    for i in range(16):
        for j in range(D // 16):
            zb[i, pl.ds(j * 16, 16)] = jnp.zeros((16,), jnp.float32)
    for k in range(RPT // 16):
        pltpu.sync_copy(zb, acc_sh.at[pl.ds(s * RPT + k * 16, 16)])
    plsc.subcore_barrier()
    bufs = (buf0, buf1)
    sems = (sem0, sem1)

    def load_iblk(bb):
        slot = bb % 2
        return (pltpu.async_copy(
                    row_hbm.at[wid, pl.ds(bb * IBLK, IBLK)], row_v.at[slot], semr),
                pltpu.async_copy(
                    col_hbm.at[wid, pl.ds(bb * IBLK, IBLK)], col_v.at[slot], semc))

    nblk = NITER // IBLK
    idx_pending = load_iblk(0)
    gather_descs = {}
    for bb in range(nblk):
        slot = bb % 2
        for dsc in idx_pending:
            dsc.wait()
        rv = row_v.at[slot]
        for i in range(IBLK):
            g = bb * IBLK + i
            gather_descs[g] = pltpu.async_copy(
                hp_hbm.at[rv.at[i]], bufs[g % 2], sems[g % 2])
            if g > 0:
                prev = g - 1
                gather_descs.pop(prev).wait()
                pcv = col_v.at[(prev // IBLK) % 2].at[prev % IBLK]
                pltpu.sync_copy(bufs[prev % 2], acc_sh.at[pcv], add=True)
            if i == 0 and bb + 1 < nblk:
                idx_pending = load_iblk(bb + 1)
    last = NITER - 1
    gather_descs.pop(last).wait()
    pltpu.sync_copy(bufs[last % 2],
                    acc_sh.at[col_v.at[(last // IBLK) % 2].at[last % IBLK]],
                    add=True)
    plsc.subcore_barrier()

    @pl.when(s < NS - 1)
    def _():
        pltpu.sync_copy(acc_sh.at[pl.ds(s * RPT, RPT)],
                        out_hbm.at[c, pl.ds(s * RPT, RPT)])

    @pl.when(s == NS - 1)
    def _():
        tail = N - (NS - 1) * RPT
        pltpu.sync_copy(acc_sh.at[pl.ds((NS - 1) * RPT, tail)],
                        out_hbm.at[c, pl.ds((NS - 1) * RPT, tail)])


def _hprime_body(x_ref, w_ref, deg_ref, hp_ref):
    dt = deg_ref[...]
    dis = lax.rsqrt(dt[:, 0] + dt[:, 1] + 1.0)
    h = jnp.dot(x_ref[...], w_ref[...], preferred_element_type=jnp.float32)
    hp_ref[...] = h * dis[:, None]


def _hprime(x, W, deg):
    blk = 1000
    return pl.pallas_call(
        _hprime_body,
        grid=(N // blk,),
        in_specs=[
            pl.BlockSpec((blk, D), lambda i: (i, 0)),
            pl.BlockSpec((D, D), lambda i: (0, 0)),
            pl.BlockSpec((blk, NC), lambda i: (i, 0)),
        ],
        out_specs=pl.BlockSpec((blk, D), lambda i: (i, 0)),
        out_shape=jax.ShapeDtypeStruct((N, D), jnp.float32),
    )(x, W, deg)


def _combine_body(acc_ref, hp_ref, deg_ref, b_ref, out_ref, stats_ref):
    i = pl.program_id(0)
    dt = deg_ref[...]
    dis = lax.rsqrt(dt[:, 0] + dt[:, 1] + 1.0)
    t = (acc_ref[0] + acc_ref[1] + hp_ref[...]) * dis[:, None] + b_ref[...]
    out_ref[...] = t
    blockstats = jnp.stack([jnp.sum(t, axis=0), jnp.sum(t * t, axis=0)])

    @pl.when(i == 0)
    def _():
        stats_ref[...] = blockstats

    @pl.when(i > 0)
    def _():
        stats_ref[...] += blockstats


def _combine(acc, hp, deg, b2):
    blk = 1000
    return pl.pallas_call(
        _combine_body,
        grid=(N // blk,),
        in_specs=[
            pl.BlockSpec((NC, blk, D), lambda i: (0, i, 0)),
            pl.BlockSpec((blk, D), lambda i: (i, 0)),
            pl.BlockSpec((blk, NC), lambda i: (i, 0)),
            pl.BlockSpec((1, D), lambda i: (0, 0)),
        ],
        out_specs=[
            pl.BlockSpec((blk, D), lambda i: (i, 0)),
            pl.BlockSpec((2, D), lambda i: (0, 0)),
        ],
        out_shape=[
            jax.ShapeDtypeStruct((N, D), jnp.float32),
            jax.ShapeDtypeStruct((2, D), jnp.float32),
        ],
    )(acc, hp, deg, b2)


def _bn_body(out_ref, stats_ref, gamma_ref, beta_ref, y_ref):
    mean = stats_ref[0, :] * (1.0 / N)
    var = stats_ref[1, :] * (1.0 / N) - mean * mean
    scale = lax.rsqrt(var + 1e-5) * gamma_ref[0, :]
    t = (out_ref[...] - mean[None, :]) * scale[None, :] + beta_ref[...]
    y_ref[...] = jnp.where(t >= 0, t, 0.01 * t)


def _bn(out_u, stats, gamma2, beta2):
    blk = 1000
    return pl.pallas_call(
        _bn_body,
        grid=(N // blk,),
        in_specs=[
            pl.BlockSpec((blk, D), lambda i: (i, 0)),
            pl.BlockSpec((2, D), lambda i: (0, 0)),
            pl.BlockSpec((1, D), lambda i: (0, 0)),
            pl.BlockSpec((1, D), lambda i: (0, 0)),
        ],
        out_specs=pl.BlockSpec((blk, D), lambda i: (i, 0)),
        out_shape=jax.ShapeDtypeStruct((N, D), jnp.float32),
    )(out_u, stats, gamma2, beta2)


@jax.jit
def kernel(x, edge_index, W, b, gamma, beta):
    row = edge_index[0].astype(jnp.int32).reshape(NW, NITER, CHUNK)
    col = edge_index[1].astype(jnp.int32).reshape(NW, NITER, CHUNK)
    deg = _deg_kernel(col)[:, :N].T
    hp = _hprime(x, W, deg)
    acc = _msg_kernel(row, col, hp)
    out_u, stats = _combine(acc, hp, deg, b.reshape(1, D))
    return _bn(out_u, stats, gamma.reshape(1, D), beta.reshape(1, D))


# speedup vs baseline: 37.1549x; 1.0640x over previous
import functools

import jax
import jax.numpy as jnp
from jax import lax
from jax.experimental import pallas as pl
from jax.experimental.pallas import tpu as pltpu
from jax.experimental.pallas import tpu_sc as plsc

N = 10000
E = 320000
D = 128
NC = 2
NS = 16
NW = NC * NS
EPW = E // NW
CHUNK = 50
NITER = EPW // CHUNK
IBLK = 8
NB = 4
NPAD = 10240
RPT = NPAD // NS

_mesh = plsc.VectorSubcoreMesh(
    core_axis_name="c", subcore_axis_name="s", num_cores=NC, num_subcores=NS)


@functools.partial(
    pl.kernel,
    out_type=jax.ShapeDtypeStruct((NC, NPAD), jnp.float32),
    mesh=_mesh,
    scratch_types=[
        pltpu.VMEM((NITER, CHUNK), jnp.int32),
        pltpu.VMEM((64,), jnp.float32),
        pltpu.VMEM((RPT,), jnp.float32),
        pltpu.VMEM_SHARED((NPAD,), jnp.float32),
        pltpu.SemaphoreType.DMA,
    ],
)
def _deg_kernel(col_hbm, deg_hbm, col_v, ones_v, zf_v, deg_sh, sem):
    c = lax.axis_index("c")
    s = lax.axis_index("s")
    wid = c * NS + s
    pltpu.sync_copy(col_hbm.at[wid], col_v)
    for j in range(64 // 16):
        ones_v[pl.ds(j * 16, 16)] = jnp.ones((16,), jnp.float32)
    for j in range(RPT // 16):
        zf_v[pl.ds(j * 16, 16)] = jnp.zeros((16,), jnp.float32)
    pltpu.sync_copy(zf_v, deg_sh.at[pl.ds(s * RPT, RPT)])
    plsc.subcore_barrier()
    ones_c = ones_v.at[pl.ds(0, CHUNK)]
    descs = []
    for i in range(NITER):
        descs.append(
            pltpu.async_copy(ones_c, deg_sh.at[col_v.at[i]], sem, add=True))
    for d in descs:
        d.wait()
    plsc.subcore_barrier()
    pltpu.sync_copy(deg_sh.at[pl.ds(s * RPT, RPT)],
                    deg_hbm.at[c, pl.ds(s * RPT, RPT)])


@functools.partial(
    pl.kernel,
    out_type=jax.ShapeDtypeStruct((NC, N, D), jnp.float32),
    mesh=_mesh,
    scratch_types=[
        pltpu.VMEM((3, IBLK, CHUNK), jnp.int32),
        pltpu.VMEM((3, IBLK, CHUNK), jnp.int32),
        pltpu.VMEM((NB, CHUNK, D), jnp.float32),
        pltpu.VMEM((16, D), jnp.float32),
        pltpu.VMEM_SHARED((NPAD, D), jnp.float32),
        [pltpu.SemaphoreType.DMA] * NB,
        [pltpu.SemaphoreType.DMA] * NB,
        pltpu.SemaphoreType.DMA,
        pltpu.SemaphoreType.DMA,
    ],
)
def _msg_kernel(row_hbm, col_hbm, hp_hbm, out_hbm,
                row_v, col_v, bufs, zb, acc_sh, gsems, ssems, semr, semc):
    c = lax.axis_index("c")
    s = lax.axis_index("s")
    wid = c * NS + s
    for i in range(16):
        for j in range(D // 16):
            zb[i, pl.ds(j * 16, 16)] = jnp.zeros((16,), jnp.float32)
    for k in range(RPT // 16):
        pltpu.sync_copy(zb, acc_sh.at[pl.ds(s * RPT + k * 16, 16)])
    plsc.subcore_barrier()

    def load_iblk(bb):
        slot = bb % 3
        return (pltpu.async_copy(
                    row_hbm.at[wid, pl.ds(bb * IBLK, IBLK)], row_v.at[slot], semr),
                pltpu.async_copy(
                    col_hbm.at[wid, pl.ds(bb * IBLK, IBLK)], col_v.at[slot], semc))

    nblk = NITER // IBLK
    idx_pending = load_iblk(0)
    gd, sd = {}, {}
    for bb in range(nblk):
        for dsc in idx_pending:
            dsc.wait()
        rv = row_v.at[bb % 3]
        for i in range(IBLK):
            g = bb * IBLK + i
            if g >= NB:
                sd.pop(g - NB).wait()
            gd[g] = pltpu.async_copy(
                hp_hbm.at[rv.at[i]], bufs.at[g % NB], gsems[g % NB])
            if g > 0:
                p = g - 1
                gd.pop(p).wait()
                pcv = col_v.at[(p // IBLK) % 3].at[p % IBLK]
                sd[p] = pltpu.async_copy(
                    bufs.at[p % NB], acc_sh.at[pcv], ssems[p % NB], add=True)
            if i == 0 and bb + 1 < nblk:
                idx_pending = load_iblk(bb + 1)
    last = NITER - 1
    gd.pop(last).wait()
    sd[last] = pltpu.async_copy(
        bufs.at[last % NB],
        acc_sh.at[col_v.at[(last // IBLK) % 3].at[last % IBLK]],
        ssems[last % NB], add=True)
    for p in sorted(sd):
        sd.pop(p).wait()
    plsc.subcore_barrier()

    @pl.when(s < NS - 1)
    def _():
        pltpu.sync_copy(acc_sh.at[pl.ds(s * RPT, RPT)],
                        out_hbm.at[c, pl.ds(s * RPT, RPT)])

    @pl.when(s == NS - 1)
    def _():
        tail = N - (NS - 1) * RPT
        pltpu.sync_copy(acc_sh.at[pl.ds((NS - 1) * RPT, tail)],
                        out_hbm.at[c, pl.ds((NS - 1) * RPT, tail)])


def _hprime_body(x_ref, w_ref, deg_ref, hp_ref):
    dt = deg_ref[...]
    dis = lax.rsqrt(dt[:, 0] + dt[:, 1] + 1.0)
    h = jnp.dot(x_ref[...], w_ref[...], preferred_element_type=jnp.float32)
    hp_ref[...] = h * dis[:, None]


def _hprime(x, W, deg):
    blk = 1000
    return pl.pallas_call(
        _hprime_body,
        grid=(N // blk,),
        in_specs=[
            pl.BlockSpec((blk, D), lambda i: (i, 0)),
            pl.BlockSpec((D, D), lambda i: (0, 0)),
            pl.BlockSpec((blk, NC), lambda i: (i, 0)),
        ],
        out_specs=pl.BlockSpec((blk, D), lambda i: (i, 0)),
        out_shape=jax.ShapeDtypeStruct((N, D), jnp.float32),
    )(x, W, deg)


def _combine_body(acc_ref, hp_ref, deg_ref, b_ref, out_ref, stats_ref):
    i = pl.program_id(0)
    dt = deg_ref[...]
    dis = lax.rsqrt(dt[:, 0] + dt[:, 1] + 1.0)
    t = (acc_ref[0] + acc_ref[1] + hp_ref[...]) * dis[:, None] + b_ref[...]
    out_ref[...] = t
    blockstats = jnp.stack([jnp.sum(t, axis=0), jnp.sum(t * t, axis=0)])

    @pl.when(i == 0)
    def _():
        stats_ref[...] = blockstats

    @pl.when(i > 0)
    def _():
        stats_ref[...] += blockstats


def _combine(acc, hp, deg, b2):
    blk = 1000
    return pl.pallas_call(
        _combine_body,
        grid=(N // blk,),
        in_specs=[
            pl.BlockSpec((NC, blk, D), lambda i: (0, i, 0)),
            pl.BlockSpec((blk, D), lambda i: (i, 0)),
            pl.BlockSpec((blk, NC), lambda i: (i, 0)),
            pl.BlockSpec((1, D), lambda i: (0, 0)),
        ],
        out_specs=[
            pl.BlockSpec((blk, D), lambda i: (i, 0)),
            pl.BlockSpec((2, D), lambda i: (0, 0)),
        ],
        out_shape=[
            jax.ShapeDtypeStruct((N, D), jnp.float32),
            jax.ShapeDtypeStruct((2, D), jnp.float32),
        ],
    )(acc, hp, deg, b2)


def _bn_body(out_ref, stats_ref, gamma_ref, beta_ref, y_ref):
    mean = stats_ref[0, :] * (1.0 / N)
    var = stats_ref[1, :] * (1.0 / N) - mean * mean
    scale = lax.rsqrt(var + 1e-5) * gamma_ref[0, :]
    t = (out_ref[...] - mean[None, :]) * scale[None, :] + beta_ref[...]
    y_ref[...] = jnp.where(t >= 0, t, 0.01 * t)


def _bn(out_u, stats, gamma2, beta2):
    blk = 1000
    return pl.pallas_call(
        _bn_body,
        grid=(N // blk,),
        in_specs=[
            pl.BlockSpec((blk, D), lambda i: (i, 0)),
            pl.BlockSpec((2, D), lambda i: (0, 0)),
            pl.BlockSpec((1, D), lambda i: (0, 0)),
            pl.BlockSpec((1, D), lambda i: (0, 0)),
        ],
        out_specs=pl.BlockSpec((blk, D), lambda i: (i, 0)),
        out_shape=jax.ShapeDtypeStruct((N, D), jnp.float32),
    )(out_u, stats, gamma2, beta2)


@jax.jit
def kernel(x, edge_index, W, b, gamma, beta):
    row = edge_index[0].astype(jnp.int32).reshape(NW, NITER, CHUNK)
    col = edge_index[1].astype(jnp.int32).reshape(NW, NITER, CHUNK)
    deg = _deg_kernel(col)[:, :N].T
    hp = _hprime(x, W, deg)
    acc = _msg_kernel(row, col, hp)
    out_u, stats = _combine(acc, hp, deg, b.reshape(1, D))
    return _bn(out_u, stats, gamma.reshape(1, D), beta.reshape(1, D))
